# baseline (speedup 1.0000x reference)
import functools

import jax
import jax.numpy as jnp
from jax import lax
from jax.experimental import pallas as pl
from jax.experimental.pallas import tpu as pltpu
from jax.experimental.pallas import tpu_sc as plsc

VOCAB = 1000000
EMBED_DIM = 64
BATCH = 4096
HIST = 200

ROW_PAD = 128
VCHUNK = 128
NBUF_G = 4
NBUF_T = 2
LAG = 2


@functools.cache
def _build():
    info = plsc.get_sparse_core_info()
    nc = info.num_cores
    nw = nc * info.num_subcores
    mesh = plsc.VectorSubcoreMesh(core_axis_name="c", subcore_axis_name="s")

    n_full = VOCAB // VCHUNK
    per_w = n_full // nw
    n_even = per_w * nw
    v_tail = n_full * VCHUNK
    n_tail = VOCAB - v_tail

    @functools.partial(
        pl.kernel,
        mesh=mesh,
        out_type=jax.ShapeDtypeStruct((VOCAB, ROW_PAD), jnp.float32),
        scratch_types=[
            pltpu.VMEM((2, EMBED_DIM, VCHUNK), jnp.float32),
            pltpu.VMEM((2, VCHUNK, ROW_PAD), jnp.float32),
        ]
        + [pltpu.SemaphoreType.DMA] * 4,
        compiler_params=pltpu.CompilerParams(needs_layout_passes=False),
    )
    def transpose_kernel(wvt_hbm, tail_hbm, scr_hbm, sbuf, buf, *sems):
        rsem = sems[:2]
        wsem = sems[2:]
        wid = lax.axis_index("s") * nc + lax.axis_index("c")
        base = jnp.arange(16, dtype=jnp.int32)

        def fire_read(v0, n, b):
            pltpu.async_copy(
                wvt_hbm.at[:, pl.ds(v0, n)], sbuf.at[b], rsem[b]
            )

        def drain_read(n, b):
            pltpu.make_async_copy(
                scr_hbm.at[pl.ds(0, n // 2)],
                buf.at[b, pl.ds(0, n // 2)],
                rsem[b],
            ).wait()

        def transpose(b):
            src = sbuf.at[b]
            dst = buf.at[b]
            diag = [(base + k) % 16 for k in range(16)]

            def vloop(vb, _):
                v0 = vb * 16
                for d0 in range(0, EMBED_DIM, 16):
                    rows = base + d0
                    for k in range(16):
                        vals = plsc.load_gather(src, [rows, diag[k] + v0])
                        plsc.store_scatter(dst, [diag[k] + v0, rows], vals)
                return ()

            lax.fori_loop(0, VCHUNK // 16, vloop, ())

        def fire_write(v0, n, b):
            pltpu.async_copy(
                buf.at[b, pl.ds(0, n)], scr_hbm.at[pl.ds(v0, n)], wsem[b]
            )

        def drain_write(n, b):
            pltpu.make_async_copy(
                scr_hbm.at[pl.ds(0, n)], buf.at[b, pl.ds(0, n)], wsem[b]
            ).wait()

        def v_of(i):
            return (i * nw + wid) * VCHUNK

        fire_read(v_of(0), VCHUNK, 0)
        fire_read(v_of(1), VCHUNK, 1)
        drain_read(VCHUNK, 0)
        transpose(0)
        fire_write(v_of(0), VCHUNK, 0)

        def slot(i, b, ob):
            drain_write(VCHUNK, b)
            fire_read(v_of(i), VCHUNK, b)
            drain_read(VCHUNK, ob)
            transpose(ob)
            fire_write(v_of(i - 1), VCHUNK, ob)

        def outer(m, _):
            i0 = 2 + m * 2
            slot(i0, 0, 1)
            slot(i0 + 1, 1, 0)
            return ()

        lax.fori_loop(0, (per_w - 2) // 2, outer, ())
        drain_read(VCHUNK, 1)
        transpose(1)
        fire_write(v_of(per_w - 1), VCHUNK, 1)
        drain_write(VCHUNK, 0)
        drain_write(VCHUNK, 1)

        @pl.when(wid < n_full - n_even)
        def _():
            v0 = (n_even + wid) * VCHUNK
            fire_read(v0, VCHUNK, 0)
            drain_read(VCHUNK, 0)
            transpose(0)
            fire_write(v0, VCHUNK, 0)
            drain_write(VCHUNK, 0)

        @pl.when(wid == 4)
        def _():
            pltpu.sync_copy(tail_hbm, buf.at[0, pl.ds(0, n_tail)])
            pltpu.sync_copy(
                buf.at[0, pl.ds(0, n_tail)], scr_hbm.at[pl.ds(v_tail, n_tail)]
            )

    b_per_w = BATCH // nw

    @functools.partial(
        pl.kernel,
        mesh=mesh,
        out_type=jax.ShapeDtypeStruct((HIST, EMBED_DIM, BATCH), jnp.float32),
        scratch_types=[
            pltpu.VMEM((HIST, b_per_w), jnp.int32),
            pltpu.VMEM((NBUF_G, b_per_w, ROW_PAD), jnp.float32),
            pltpu.VMEM((NBUF_T, EMBED_DIM, b_per_w), jnp.float32),
        ]
        + [pltpu.SemaphoreType.DMA] * (NBUF_G + NBUF_T),
        compiler_params=pltpu.CompilerParams(needs_layout_passes=False),
    )
    def gather_kernel(xt_hbm, scr_hbm, out_hbm, idx_v, gbuf, tbuf, *sems):
        gsem = sems[:NBUF_G]
        wsem = sems[NBUF_G:]
        wid = lax.axis_index("s") * nc + lax.axis_index("c")
        b0 = wid * b_per_w

        pltpu.sync_copy(xt_hbm.at[:, pl.ds(b0, b_per_w)], idx_v)

        def fire_gather(h, g):
            pltpu.async_copy(
                scr_hbm.at[idx_v.at[h]], gbuf.at[g], gsem[g]
            )

        def drain_gather(g):
            pltpu.make_async_copy(
                scr_hbm.at[pl.ds(0, b_per_w)], gbuf.at[g], gsem[g]
            ).wait()

        base = jnp.arange(16, dtype=jnp.int32)
        diag = [(base + k) % 16 for k in range(16)]

        def transpose(g, t):
            src = gbuf.at[g]
            dst = tbuf.at[t]

            def bloop(bb, _):
                b00 = bb * 16
                for d0 in range(0, EMBED_DIM, 16):
                    for k in range(16):
                        vals = plsc.load_gather(
                            src, [base + b00, diag[k] + d0]
                        )
                        plsc.store_scatter(
                            dst, [diag[k] + d0, base + b00], vals
                        )
                return ()

            lax.fori_loop(0, b_per_w // 16, bloop, ())

        def fire_write(h, t):
            pltpu.async_copy(
                tbuf.at[t], out_hbm.at[h, :, pl.ds(b0, b_per_w)], wsem[t]
            )

        def drain_write(t):
            pltpu.make_async_copy(
                scr_hbm.at[pl.ds(0, EMBED_DIM), pl.ds(0, b_per_w)],
                tbuf.at[t],
                wsem[t],
            ).wait()

        def slot(h, g, t, late, refire):
            drain_gather(g)
            if late:
                drain_write(t)
            transpose(g, t)
            fire_write(h, t)
            if refire:
                fire_gather(h + NBUF_G, g)

        for g in range(NBUF_G):
            fire_gather(g, g)
        slot(0, 0, 0, False, True)
        slot(1, 1, 1, False, True)

        def outer(m, _):
            h0 = 2 + m * NBUF_G
            for k in range(NBUF_G):
                slot(h0 + k, (2 + k) % NBUF_G, k % NBUF_T, True, True)
            return ()

        lax.fori_loop(0, (HIST - 2 - NBUF_G) // NBUF_G, outer, ())

        for h in range(HIST - 2 - NBUF_G, HIST):
            slot(h, h % NBUF_G, h % NBUF_T, True, h + NBUF_G < HIST)
        drain_write(0)
        drain_write(1)

    return transpose_kernel, gather_kernel


def kernel(x, word_vectors):
    transpose_kernel, gather_kernel = _build()
    wvt = word_vectors.T
    xt = x.T
    v_tail = (VOCAB // VCHUNK) * VCHUNK
    tail = jnp.pad(
        word_vectors[v_tail:], ((0, 0), (0, ROW_PAD - EMBED_DIM))
    )
    scratch = transpose_kernel(wvt, tail)
    out_t = gather_kernel(xt, scratch)
    return jnp.transpose(out_t, (2, 0, 1))


# speedup vs baseline: 1.0565x; 1.0565x over previous
import functools

import jax
import jax.numpy as jnp
from jax import lax
from jax.experimental import pallas as pl
from jax.experimental.pallas import tpu as pltpu
from jax.experimental.pallas import tpu_sc as plsc

VOCAB = 1000000
EMBED_DIM = 64
BATCH = 4096
HIST = 200

ROW_PAD = 128
VCHUNK = 256
NBUF_G = 4
NBUF_T = 2
LAG = 2


@functools.cache
def _build():
    info = plsc.get_sparse_core_info()
    nc = info.num_cores
    nw = nc * info.num_subcores
    mesh = plsc.VectorSubcoreMesh(core_axis_name="c", subcore_axis_name="s")

    n_full = VOCAB // VCHUNK
    per_w = n_full // nw
    n_even = per_w * nw
    v_tail = n_full * VCHUNK
    n_tail = VOCAB - v_tail

    @functools.partial(
        pl.kernel,
        mesh=mesh,
        out_type=jax.ShapeDtypeStruct((VOCAB, ROW_PAD), jnp.float32),
        scratch_types=[
            pltpu.VMEM((2, EMBED_DIM, VCHUNK), jnp.float32),
            pltpu.VMEM((2, VCHUNK, ROW_PAD), jnp.float32),
        ]
        + [pltpu.SemaphoreType.DMA] * 4,
        compiler_params=pltpu.CompilerParams(needs_layout_passes=False),
    )
    def transpose_kernel(wvt_hbm, tail_hbm, scr_hbm, sbuf, buf, *sems):
        rsem = sems[:2]
        wsem = sems[2:]
        wid = lax.axis_index("s") * nc + lax.axis_index("c")
        base = jnp.arange(16, dtype=jnp.int32)

        def fire_read(v0, n, b):
            pltpu.async_copy(
                wvt_hbm.at[:, pl.ds(v0, n)], sbuf.at[b], rsem[b]
            )

        def drain_read(n, b):
            pltpu.make_async_copy(
                scr_hbm.at[pl.ds(0, n // 2)],
                buf.at[b, pl.ds(0, n // 2)],
                rsem[b],
            ).wait()

        def transpose(b):
            src = sbuf.at[b]
            dst = buf.at[b]
            diag = [(base + k) % 16 for k in range(16)]

            def vloop(vb, _):
                v0 = vb * 16
                for d0 in range(0, EMBED_DIM, 16):
                    rows = base + d0
                    for k in range(16):
                        vals = plsc.load_gather(src, [rows, diag[k] + v0])
                        plsc.store_scatter(dst, [diag[k] + v0, rows], vals)
                return ()

            lax.fori_loop(0, VCHUNK // 16, vloop, ())

        def fire_write(v0, n, b):
            pltpu.async_copy(
                buf.at[b, pl.ds(0, n)], scr_hbm.at[pl.ds(v0, n)], wsem[b]
            )

        def drain_write(n, b):
            pltpu.make_async_copy(
                scr_hbm.at[pl.ds(0, n)], buf.at[b, pl.ds(0, n)], wsem[b]
            ).wait()

        def v_of(i):
            return (i * nw + wid) * VCHUNK

        fire_read(v_of(0), VCHUNK, 0)
        fire_read(v_of(1), VCHUNK, 1)
        drain_read(VCHUNK, 0)
        transpose(0)
        fire_write(v_of(0), VCHUNK, 0)

        def slot(i, b, ob):
            drain_write(VCHUNK, b)
            fire_read(v_of(i), VCHUNK, b)
            drain_read(VCHUNK, ob)
            transpose(ob)
            fire_write(v_of(i - 1), VCHUNK, ob)

        def outer(m, _):
            i0 = 2 + m * 2
            slot(i0, 0, 1)
            slot(i0 + 1, 1, 0)
            return ()

        lax.fori_loop(0, (per_w - 2) // 2, outer, ())
        drain_read(VCHUNK, 1)
        transpose(1)
        fire_write(v_of(per_w - 1), VCHUNK, 1)
        drain_write(VCHUNK, 0)
        drain_write(VCHUNK, 1)

        @pl.when(wid < n_full - n_even)
        def _():
            v0 = (n_even + wid) * VCHUNK
            fire_read(v0, VCHUNK, 0)
            drain_read(VCHUNK, 0)
            transpose(0)
            fire_write(v0, VCHUNK, 0)
            drain_write(VCHUNK, 0)

        @pl.when(wid == 4)
        def _():
            pltpu.sync_copy(tail_hbm, buf.at[0, pl.ds(0, n_tail)])
            pltpu.sync_copy(
                buf.at[0, pl.ds(0, n_tail)], scr_hbm.at[pl.ds(v_tail, n_tail)]
            )

    b_per_w = BATCH // nw

    @functools.partial(
        pl.kernel,
        mesh=mesh,
        out_type=jax.ShapeDtypeStruct((HIST, EMBED_DIM, BATCH), jnp.float32),
        scratch_types=[
            pltpu.VMEM((HIST, b_per_w), jnp.int32),
            pltpu.VMEM((NBUF_G, b_per_w, ROW_PAD), jnp.float32),
            pltpu.VMEM((NBUF_T, EMBED_DIM, b_per_w), jnp.float32),
        ]
        + [pltpu.SemaphoreType.DMA] * (NBUF_G + NBUF_T),
        compiler_params=pltpu.CompilerParams(needs_layout_passes=False),
    )
    def gather_kernel(xt_hbm, scr_hbm, out_hbm, idx_v, gbuf, tbuf, *sems):
        gsem = sems[:NBUF_G]
        wsem = sems[NBUF_G:]
        wid = lax.axis_index("s") * nc + lax.axis_index("c")
        b0 = wid * b_per_w

        pltpu.sync_copy(xt_hbm.at[:, pl.ds(b0, b_per_w)], idx_v)

        def fire_gather(h, g):
            pltpu.async_copy(
                scr_hbm.at[idx_v.at[h]], gbuf.at[g], gsem[g]
            )

        def drain_gather(g):
            pltpu.make_async_copy(
                scr_hbm.at[pl.ds(0, b_per_w)], gbuf.at[g], gsem[g]
            ).wait()

        base = jnp.arange(16, dtype=jnp.int32)
        diag = [(base + k) % 16 for k in range(16)]

        def transpose(g, t):
            src = gbuf.at[g]
            dst = tbuf.at[t]

            def bloop(bb, _):
                b00 = bb * 16
                for d0 in range(0, EMBED_DIM, 16):
                    for k in range(16):
                        vals = plsc.load_gather(
                            src, [base + b00, diag[k] + d0]
                        )
                        plsc.store_scatter(
                            dst, [diag[k] + d0, base + b00], vals
                        )
                return ()

            lax.fori_loop(0, b_per_w // 16, bloop, ())

        def fire_write(h, t):
            pltpu.async_copy(
                tbuf.at[t], out_hbm.at[h, :, pl.ds(b0, b_per_w)], wsem[t]
            )

        def drain_write(t):
            pltpu.make_async_copy(
                scr_hbm.at[pl.ds(0, EMBED_DIM), pl.ds(0, b_per_w)],
                tbuf.at[t],
                wsem[t],
            ).wait()

        def slot(h, g, t, late, refire):
            drain_gather(g)
            if late:
                drain_write(t)
            transpose(g, t)
            fire_write(h, t)
            if refire:
                fire_gather(h + NBUF_G, g)

        for g in range(NBUF_G):
            fire_gather(g, g)
        slot(0, 0, 0, False, True)
        slot(1, 1, 1, False, True)

        def outer(m, _):
            h0 = 2 + m * NBUF_G
            for k in range(NBUF_G):
                slot(h0 + k, (2 + k) % NBUF_G, k % NBUF_T, True, True)
            return ()

        lax.fori_loop(0, (HIST - 2 - NBUF_G) // NBUF_G, outer, ())

        for h in range(HIST - 2 - NBUF_G, HIST):
            slot(h, h % NBUF_G, h % NBUF_T, True, h + NBUF_G < HIST)
        drain_write(0)
        drain_write(1)

    return transpose_kernel, gather_kernel


def kernel(x, word_vectors):
    transpose_kernel, gather_kernel = _build()
    wvt = word_vectors.T
    xt = x.T
    v_tail = (VOCAB // VCHUNK) * VCHUNK
    tail = jnp.pad(
        word_vectors[v_tail:], ((0, 0), (0, ROW_PAD - EMBED_DIM))
    )
    scratch = transpose_kernel(wvt, tail)
    out_t = gather_kernel(xt, scratch)
    return jnp.transpose(out_t, (2, 0, 1))
